# Initial kernel scaffold; baseline (speedup 1.0000x reference)
import jax
import jax.numpy as jnp
from jax.experimental import pallas as pl


def kernel(x, edge_index, W1, b1, W2, b2):
    raise NotImplementedError("write your pallas kernel here")


# speedup vs baseline: 3.7195x; 3.7195x over previous
import functools

import jax
import jax.numpy as jnp
from jax import lax
from jax.experimental import pallas as pl
from jax.experimental.pallas import tpu as pltpu
from jax.experimental.pallas import tpu_sc as plsc

N = 10000
D = 768
E = 100000

NTILE = 16
NPAD = 10240
ROWS_PER_TILE = NPAD // NTILE
NCHUNK = 6
CW = D // NCHUNK
EPAD = 102400
EB = 128
EDGES_PER_TILE = EPAD // NTILE
NBATCH = EDGES_PER_TILE // EB
DEGW = 128


def _sc_mesh():
    return plsc.VectorSubcoreMesh(core_axis_name="c", subcore_axis_name="s")


DEG_EPC = EPAD // 2
DEG_EPT = DEG_EPC // NTILE
DEG_NB = DEG_EPT // EB


def _deg_body(dst_hbm, ones_hbm, zeros_hbm, out_hbm, acc_sh, dstv, onesv, semv):
    cid = lax.axis_index("c")
    sid = lax.axis_index("s")
    r0 = sid * ROWS_PER_TILE
    pltpu.sync_copy(zeros_hbm.at[pl.ds(r0, ROWS_PER_TILE)], acc_sh.at[pl.ds(r0, ROWS_PER_TILE)])
    pltpu.sync_copy(ones_hbm, onesv)
    plsc.subcore_barrier()
    def body(j, carry):
        off = cid * DEG_EPC + sid * DEG_EPT + j * EB
        pltpu.sync_copy(dst_hbm.at[pl.ds(off, EB)], dstv)
        pltpu.sync_copy(onesv, acc_sh.at[dstv], add=True)
        return carry
    lax.fori_loop(0, DEG_NB, body, 0)
    plsc.subcore_barrier()
    pltpu.sync_copy(acc_sh.at[pl.ds(r0, ROWS_PER_TILE)],
                    out_hbm.at[cid].at[pl.ds(r0, ROWS_PER_TILE)])


def _sc_deg(dstp, ones_b, zeros_b):
    k = functools.partial(
        pl.kernel,
        out_type=jax.ShapeDtypeStruct((2, NPAD, DEGW), jnp.float32),
        mesh=_sc_mesh(),
        scratch_types=[
            pltpu.VMEM_SHARED((NPAD, DEGW), jnp.float32),
            pltpu.VMEM((EB,), jnp.int32),
            pltpu.VMEM((EB, DEGW), jnp.float32),
            pltpu.SemaphoreType.DMA,
        ],
    )(_deg_body)
    return k(dstp, ones_b, zeros_b)


def _agg_body(hp2d_hbm, hp4_hbm, src_hbm, dst_hbm, out_hbm,
              acc_sh, srcv, dstv, gidxv, rows, sem):
    cid = lax.axis_index("c")
    sid = lax.axis_index("s")
    r0 = sid * ROWS_PER_TILE
    for c_local in range(NCHUNK // 2):
        chunk = cid * (NCHUNK // 2) + c_local
        colpos = chunk * CW
        pltpu.sync_copy(hp2d_hbm.at[pl.ds(r0, ROWS_PER_TILE), pl.ds(colpos, CW)],
                        acc_sh.at[pl.ds(r0, ROWS_PER_TILE)])
        plsc.subcore_barrier()

        def body(j, carry):
            off = sid * EDGES_PER_TILE + j * EB
            pltpu.sync_copy(src_hbm.at[pl.ds(off, EB)], srcv)
            pltpu.sync_copy(dst_hbm.at[pl.ds(off, EB)], dstv)
            for i in range(EB // 16):
                s16 = srcv[pl.ds(i * 16, 16)]
                gidxv[pl.ds(i * 16, 16)] = s16 * NCHUNK + chunk
            pltpu.async_copy(hp4_hbm.at[gidxv], rows, sem).wait()
            pltpu.sync_copy(rows, acc_sh.at[dstv], add=True)
            return carry

        lax.fori_loop(0, NBATCH, body, 0)
        plsc.subcore_barrier()
        pltpu.sync_copy(acc_sh.at[pl.ds(r0, ROWS_PER_TILE)],
                        out_hbm.at[pl.ds(r0, ROWS_PER_TILE), pl.ds(colpos, CW)])
        plsc.subcore_barrier()


def _sc_agg(hp, srcp, dstp):
    hp4 = hp.reshape(NPAD * NCHUNK, CW)
    k = functools.partial(
        pl.kernel,
        out_type=jax.ShapeDtypeStruct((NPAD, D), jnp.float32),
        mesh=_sc_mesh(),
        scratch_types=[
            pltpu.VMEM_SHARED((NPAD, CW), jnp.float32),
            pltpu.VMEM((EB,), jnp.int32),
            pltpu.VMEM((EB,), jnp.int32),
            pltpu.VMEM((EB,), jnp.int32),
            pltpu.VMEM((EB, CW), jnp.float32),
            pltpu.SemaphoreType.DMA,
        ],
    )(_agg_body)
    return k(hp, hp4, srcp, dstp)


BM = 1024


def _dis(deg_ref):
    return lax.rsqrt(deg_ref[0, :, 0:1] + deg_ref[1, :, 0:1] + 1.0)


def _tc1_body(x_ref, w_ref, deg_ref, o_ref):
    dis = _dis(deg_ref)
    h = jnp.dot(x_ref[...], w_ref[...], preferred_element_type=jnp.float32)
    o_ref[...] = h * dis


def _tc1(xpad, W1, deg):
    return pl.pallas_call(
        _tc1_body,
        grid=(NPAD // BM,),
        in_specs=[
            pl.BlockSpec((BM, D), lambda m: (m, 0)),
            pl.BlockSpec((D, D), lambda m: (0, 0)),
            pl.BlockSpec((2, BM, DEGW), lambda m: (0, m, 0)),
        ],
        out_specs=pl.BlockSpec((BM, D), lambda m: (m, 0)),
        out_shape=jax.ShapeDtypeStruct((NPAD, D), jnp.float32),
    )(xpad, W1, deg)


def _tc2_body(a_ref, w_ref, deg_ref, b_ref, o_ref):
    dis = _dis(deg_ref)
    h = jnp.maximum(a_ref[...] * dis + b_ref[...], 0.0)
    o_ref[...] = jnp.dot(h, w_ref[...], preferred_element_type=jnp.float32) * dis


def _tc2(acc1, W2, deg, b1):
    return pl.pallas_call(
        _tc2_body,
        grid=(NPAD // BM,),
        in_specs=[
            pl.BlockSpec((BM, D), lambda m: (m, 0)),
            pl.BlockSpec((D, D), lambda m: (0, 0)),
            pl.BlockSpec((2, BM, DEGW), lambda m: (0, m, 0)),
            pl.BlockSpec((1, D), lambda m: (0, 0)),
        ],
        out_specs=pl.BlockSpec((BM, D), lambda m: (m, 0)),
        out_shape=jax.ShapeDtypeStruct((NPAD, D), jnp.float32),
    )(acc1, W2, deg, b1)


def _tc3_body(a_ref, deg_ref, b_ref, o_ref):
    dis = _dis(deg_ref)
    o_ref[...] = a_ref[...] * dis + b_ref[...]


def _tc3(acc2, deg, b2):
    return pl.pallas_call(
        _tc3_body,
        grid=(NPAD // BM,),
        in_specs=[
            pl.BlockSpec((BM, D), lambda m: (m, 0)),
            pl.BlockSpec((2, BM, DEGW), lambda m: (0, m, 0)),
            pl.BlockSpec((1, D), lambda m: (0, 0)),
        ],
        out_specs=pl.BlockSpec((BM, D), lambda m: (m, 0)),
        out_shape=jax.ShapeDtypeStruct((NPAD, D), jnp.float32),
    )(acc2, deg, b2)


def kernel(x, edge_index, W1, b1, W2, b2):
    ei = edge_index.astype(jnp.int32)
    srcp = jnp.concatenate([ei[0], jnp.zeros((EPAD - E,), jnp.int32)])
    dstp = jnp.concatenate([ei[1], jnp.full((EPAD - E,), N, jnp.int32)])
    xpad = jnp.pad(x, ((0, NPAD - N), (0, 0)))
    ones_b = jnp.ones((EB, DEGW), jnp.float32)
    zeros_b = jnp.zeros((NPAD, DEGW), jnp.float32)

    deg = _sc_deg(dstp, ones_b, zeros_b)
    h1p = _tc1(xpad, W1, deg)
    acc1 = _sc_agg(h1p, srcp, dstp)
    h2p = _tc2(acc1, W2, deg, b1.reshape(1, D))
    acc2 = _sc_agg(h2p, srcp, dstp)
    out = _tc3(acc2, deg, b2.reshape(1, D))
    return out[:N]


# speedup vs baseline: 4.7218x; 1.2695x over previous
import functools

import jax
import jax.numpy as jnp
from jax import lax
from jax.experimental import pallas as pl
from jax.experimental.pallas import tpu as pltpu
from jax.experimental.pallas import tpu_sc as plsc

N = 10000
D = 768
E = 100000

NTILE = 16
NPAD = 10240
ROWS_PER_TILE = NPAD // NTILE
NCHUNK = 6
CW = D // NCHUNK
EPAD = 102400
EB = 128
EDGES_PER_TILE = EPAD // NTILE
NBATCH = EDGES_PER_TILE // EB
DEGW = 128


def _sc_mesh():
    return plsc.VectorSubcoreMesh(core_axis_name="c", subcore_axis_name="s")


DEG_EPC = EPAD // 2
DEG_EPT = DEG_EPC // NTILE
DEG_NB = DEG_EPT // EB


def _deg_body(dst_hbm, ones_hbm, zeros_hbm, out_hbm, acc_sh, dstv, onesv, semv):
    cid = lax.axis_index("c")
    sid = lax.axis_index("s")
    r0 = sid * ROWS_PER_TILE
    pltpu.sync_copy(zeros_hbm.at[pl.ds(r0, ROWS_PER_TILE)], acc_sh.at[pl.ds(r0, ROWS_PER_TILE)])
    pltpu.sync_copy(ones_hbm, onesv)
    plsc.subcore_barrier()
    def body(j, carry):
        off = cid * DEG_EPC + sid * DEG_EPT + j * EB
        pltpu.sync_copy(dst_hbm.at[pl.ds(off, EB)], dstv)
        pltpu.sync_copy(onesv, acc_sh.at[dstv], add=True)
        return carry
    lax.fori_loop(0, DEG_NB, body, 0)
    plsc.subcore_barrier()
    pltpu.sync_copy(acc_sh.at[pl.ds(r0, ROWS_PER_TILE)],
                    out_hbm.at[cid].at[pl.ds(r0, ROWS_PER_TILE)])


def _sc_deg(dstp, ones_b, zeros_b):
    k = functools.partial(
        pl.kernel,
        out_type=jax.ShapeDtypeStruct((2, NPAD, DEGW), jnp.float32),
        mesh=_sc_mesh(),
        scratch_types=[
            pltpu.VMEM_SHARED((NPAD, DEGW), jnp.float32),
            pltpu.VMEM((EB,), jnp.int32),
            pltpu.VMEM((EB, DEGW), jnp.float32),
            pltpu.SemaphoreType.DMA,
        ],
    )(_deg_body)
    return k(dstp, ones_b, zeros_b)


def _agg_body(hp2d_hbm, hp4_hbm, src_hbm, dst_hbm, out_hbm,
              acc_sh, srcb, dstb, gidxb, rowsb,
              semG0, semG1, semI0, semI1):
    cid = lax.axis_index("c")
    sid = lax.axis_index("s")
    r0 = sid * ROWS_PER_TILE
    e0 = sid * EDGES_PER_TILE
    semG = (semG0, semG1)
    semI = (semI0, semI1)

    for c_local in range(NCHUNK // 2):
        chunk = cid * (NCHUNK // 2) + c_local
        colpos = chunk * CW

        def idx_issue(j, b):
            off = e0 + lax.rem(j, NBATCH) * EB
            pltpu.async_copy(src_hbm.at[pl.ds(off, EB)], srcb.at[b], semI[b])
            pltpu.async_copy(dst_hbm.at[pl.ds(off, EB)], dstb.at[b], semI[b])

        def idx_wait(j, b):
            off = e0 + lax.rem(j, NBATCH) * EB
            pltpu.make_async_copy(src_hbm.at[pl.ds(off, EB)], srcb.at[b], semI[b]).wait()
            pltpu.make_async_copy(dst_hbm.at[pl.ds(off, EB)], dstb.at[b], semI[b]).wait()

        def gather_issue(b):
            for i in range(EB // 16):
                s16 = srcb[b, pl.ds(i * 16, 16)]
                gidxb[b, pl.ds(i * 16, 16)] = s16 * NCHUNK + chunk
            pltpu.async_copy(hp4_hbm.at[gidxb.at[b]], rowsb.at[b], semG[b])

        def gather_wait(b):
            pltpu.make_async_copy(hp4_hbm.at[gidxb.at[b]], rowsb.at[b], semG[b]).wait()

        pltpu.sync_copy(hp2d_hbm.at[pl.ds(r0, ROWS_PER_TILE), pl.ds(colpos, CW)],
                        acc_sh.at[pl.ds(r0, ROWS_PER_TILE)])
        plsc.subcore_barrier()

        idx_issue(jnp.int32(0), 0)
        idx_wait(jnp.int32(0), 0)
        gather_issue(0)
        idx_issue(jnp.int32(1), 1)

        def pair(i, carry):
            for b in (0, 1):
                j = i * 2 + b
                o = 1 - b
                gather_wait(b)
                idx_wait(j + 1, o)
                gather_issue(o)
                pltpu.sync_copy(rowsb.at[b], acc_sh.at[dstb.at[b]], add=True)
                idx_issue(j + 2, b)
            return carry

        lax.fori_loop(0, NBATCH // 2, pair, 0)
        gather_wait(0)
        idx_wait(jnp.int32(NBATCH + 1), 1)
        plsc.subcore_barrier()
        pltpu.sync_copy(acc_sh.at[pl.ds(r0, ROWS_PER_TILE)],
                        out_hbm.at[pl.ds(r0, ROWS_PER_TILE), pl.ds(colpos, CW)])
        plsc.subcore_barrier()


def _sc_agg(hp, srcp, dstp):
    hp4 = hp.reshape(NPAD * NCHUNK, CW)
    k = functools.partial(
        pl.kernel,
        out_type=jax.ShapeDtypeStruct((NPAD, D), jnp.float32),
        mesh=_sc_mesh(),
        scratch_types=[
            pltpu.VMEM_SHARED((NPAD, CW), jnp.float32),
            pltpu.VMEM((2, EB), jnp.int32),
            pltpu.VMEM((2, EB), jnp.int32),
            pltpu.VMEM((2, EB), jnp.int32),
            pltpu.VMEM((2, EB, CW), jnp.float32),
            pltpu.SemaphoreType.DMA,
            pltpu.SemaphoreType.DMA,
            pltpu.SemaphoreType.DMA,
            pltpu.SemaphoreType.DMA,
        ],
    )(_agg_body)
    return k(hp, hp4, srcp, dstp)


BM = 1024


def _dis(deg_ref):
    return lax.rsqrt(deg_ref[0, :, 0:1] + deg_ref[1, :, 0:1] + 1.0)


def _tc1_body(x_ref, w_ref, deg_ref, o_ref):
    dis = _dis(deg_ref)
    h = jnp.dot(x_ref[...], w_ref[...], preferred_element_type=jnp.float32)
    o_ref[...] = h * dis


def _tc1(xpad, W1, deg):
    return pl.pallas_call(
        _tc1_body,
        grid=(NPAD // BM,),
        in_specs=[
            pl.BlockSpec((BM, D), lambda m: (m, 0)),
            pl.BlockSpec((D, D), lambda m: (0, 0)),
            pl.BlockSpec((2, BM, DEGW), lambda m: (0, m, 0)),
        ],
        out_specs=pl.BlockSpec((BM, D), lambda m: (m, 0)),
        out_shape=jax.ShapeDtypeStruct((NPAD, D), jnp.float32),
    )(xpad, W1, deg)


def _tc2_body(a_ref, w_ref, deg_ref, b_ref, o_ref):
    dis = _dis(deg_ref)
    h = jnp.maximum(a_ref[...] * dis + b_ref[...], 0.0)
    o_ref[...] = jnp.dot(h, w_ref[...], preferred_element_type=jnp.float32) * dis


def _tc2(acc1, W2, deg, b1):
    return pl.pallas_call(
        _tc2_body,
        grid=(NPAD // BM,),
        in_specs=[
            pl.BlockSpec((BM, D), lambda m: (m, 0)),
            pl.BlockSpec((D, D), lambda m: (0, 0)),
            pl.BlockSpec((2, BM, DEGW), lambda m: (0, m, 0)),
            pl.BlockSpec((1, D), lambda m: (0, 0)),
        ],
        out_specs=pl.BlockSpec((BM, D), lambda m: (m, 0)),
        out_shape=jax.ShapeDtypeStruct((NPAD, D), jnp.float32),
    )(acc1, W2, deg, b1)


def _tc3_body(a_ref, deg_ref, b_ref, o_ref):
    dis = _dis(deg_ref)
    o_ref[...] = a_ref[...] * dis + b_ref[...]


def _tc3(acc2, deg, b2):
    return pl.pallas_call(
        _tc3_body,
        grid=(NPAD // BM,),
        in_specs=[
            pl.BlockSpec((BM, D), lambda m: (m, 0)),
            pl.BlockSpec((2, BM, DEGW), lambda m: (0, m, 0)),
            pl.BlockSpec((1, D), lambda m: (0, 0)),
        ],
        out_specs=pl.BlockSpec((BM, D), lambda m: (m, 0)),
        out_shape=jax.ShapeDtypeStruct((NPAD, D), jnp.float32),
    )(acc2, deg, b2)


def kernel(x, edge_index, W1, b1, W2, b2):
    ei = edge_index.astype(jnp.int32)
    srcp = jnp.concatenate([ei[0], jnp.zeros((EPAD - E,), jnp.int32)])
    dstp = jnp.concatenate([ei[1], jnp.full((EPAD - E,), N, jnp.int32)])
    xpad = jnp.pad(x, ((0, NPAD - N), (0, 0)))
    ones_b = jnp.ones((EB, DEGW), jnp.float32)
    zeros_b = jnp.zeros((NPAD, DEGW), jnp.float32)

    deg = _sc_deg(dstp, ones_b, zeros_b)
    h1p = _tc1(xpad, W1, deg)
    acc1 = _sc_agg(h1p, srcp, dstp)
    h2p = _tc2(acc1, W2, deg, b1.reshape(1, D))
    acc2 = _sc_agg(h2p, srcp, dstp)
    out = _tc3(acc2, deg, b2.reshape(1, D))
    return out[:N]


# speedup vs baseline: 4.7292x; 1.0016x over previous
import functools

import jax
import jax.numpy as jnp
from jax import lax
from jax.experimental import pallas as pl
from jax.experimental.pallas import tpu as pltpu
from jax.experimental.pallas import tpu_sc as plsc

N = 10000
D = 768
E = 100000

NTILE = 16
NPAD = 10240
ROWS_PER_TILE = NPAD // NTILE
NCHUNK = 6
CW = D // NCHUNK
EPAD = 102400
EB = 128
EDGES_PER_TILE = EPAD // NTILE
NBATCH = EDGES_PER_TILE // EB
DEGW = 128


def _sc_mesh():
    return plsc.VectorSubcoreMesh(core_axis_name="c", subcore_axis_name="s")


DEG_EPC = EPAD // 2
DEG_EPT = DEG_EPC // NTILE
DEG_NB = DEG_EPT // EB


def _deg_body(dst_hbm, ones_hbm, zeros_hbm, out_hbm, acc_sh, dstv, onesv, semv):
    cid = lax.axis_index("c")
    sid = lax.axis_index("s")
    r0 = sid * ROWS_PER_TILE
    pltpu.sync_copy(zeros_hbm.at[pl.ds(r0, ROWS_PER_TILE)], acc_sh.at[pl.ds(r0, ROWS_PER_TILE)])
    pltpu.sync_copy(ones_hbm, onesv)
    plsc.subcore_barrier()
    def body(j, carry):
        off = cid * DEG_EPC + sid * DEG_EPT + j * EB
        pltpu.sync_copy(dst_hbm.at[pl.ds(off, EB)], dstv)
        pltpu.sync_copy(onesv, acc_sh.at[dstv], add=True)
        return carry
    lax.fori_loop(0, DEG_NB, body, 0)
    plsc.subcore_barrier()
    pltpu.sync_copy(acc_sh.at[pl.ds(r0, ROWS_PER_TILE)],
                    out_hbm.at[cid].at[pl.ds(r0, ROWS_PER_TILE)])


def _sc_deg(dstp, ones_b, zeros_b):
    k = functools.partial(
        pl.kernel,
        out_type=jax.ShapeDtypeStruct((2, NPAD, DEGW), jnp.float32),
        mesh=_sc_mesh(),
        scratch_types=[
            pltpu.VMEM_SHARED((NPAD, DEGW), jnp.float32),
            pltpu.VMEM((EB,), jnp.int32),
            pltpu.VMEM((EB, DEGW), jnp.float32),
            pltpu.SemaphoreType.DMA,
        ],
    )(_deg_body)
    return k(dstp, ones_b, zeros_b)


AEB = 128
ANB = EDGES_PER_TILE // AEB


def _agg_body(hp2d_hbm, hp4_hbm, src_hbm, dst_hbm, out_hbm,
              acc_sh, srcb, dstb, gidxb, rowsb,
              semG0, semG1, semI0, semI1, semS0, semS1):
    cid = lax.axis_index("c")
    sid = lax.axis_index("s")
    r0 = sid * ROWS_PER_TILE
    e0 = sid * EDGES_PER_TILE
    semG = (semG0, semG1)
    semI = (semI0, semI1)
    semS = (semS0, semS1)

    for c_local in range(NCHUNK // 2):
        chunk = cid * (NCHUNK // 2) + c_local
        colpos = chunk * CW

        def idx_issue(j, b):
            off = e0 + lax.rem(j, ANB) * AEB
            pltpu.async_copy(src_hbm.at[pl.ds(off, AEB)], srcb.at[b], semI[b])
            pltpu.async_copy(dst_hbm.at[pl.ds(off, AEB)], dstb.at[b], semI[b])

        def idx_wait(j, b):
            off = e0 + lax.rem(j, ANB) * AEB
            pltpu.make_async_copy(src_hbm.at[pl.ds(off, AEB)], srcb.at[b], semI[b]).wait()
            pltpu.make_async_copy(dst_hbm.at[pl.ds(off, AEB)], dstb.at[b], semI[b]).wait()

        def gather_issue(b):
            for i in range(AEB // 16):
                s16 = srcb[b, pl.ds(i * 16, 16)]
                gidxb[b, pl.ds(i * 16, 16)] = s16 * NCHUNK + chunk
            pltpu.async_copy(hp4_hbm.at[gidxb.at[b]], rowsb.at[b], semG[b])

        def gather_wait(b):
            pltpu.make_async_copy(hp4_hbm.at[gidxb.at[b]], rowsb.at[b], semG[b]).wait()

        def scatter_issue(b):
            pltpu.async_copy(rowsb.at[b], acc_sh.at[dstb.at[b]], semS[b], add=True)

        def scatter_wait(b):
            pltpu.make_async_copy(rowsb.at[b], acc_sh.at[dstb.at[b]], semS[b]).wait()

        def step(j, b):
            o = 1 - b
            gather_wait(b)
            scatter_issue(b)
            idx_wait(j + 1, o)
            gather_issue(o)
            scatter_wait(b)
            idx_issue(j + 2, b)

        pltpu.sync_copy(hp2d_hbm.at[pl.ds(r0, ROWS_PER_TILE), pl.ds(colpos, CW)],
                        acc_sh.at[pl.ds(r0, ROWS_PER_TILE)])
        plsc.subcore_barrier()

        idx_issue(jnp.int32(0), 0)
        idx_wait(jnp.int32(0), 0)
        gather_issue(0)
        idx_issue(jnp.int32(1), 1)

        def pair(i, carry):
            step(2 * i, 0)
            step(2 * i + 1, 1)
            return carry

        lax.fori_loop(0, ANB // 2, pair, 0)
        gather_wait(0)
        idx_wait(jnp.int32(ANB + 1), 1)
        plsc.subcore_barrier()
        pltpu.sync_copy(acc_sh.at[pl.ds(r0, ROWS_PER_TILE)],
                        out_hbm.at[pl.ds(r0, ROWS_PER_TILE), pl.ds(colpos, CW)])
        plsc.subcore_barrier()


def _sc_agg(hp, srcp, dstp):
    hp4 = hp.reshape(NPAD * NCHUNK, CW)
    k = functools.partial(
        pl.kernel,
        out_type=jax.ShapeDtypeStruct((NPAD, D), jnp.float32),
        mesh=_sc_mesh(),
        scratch_types=[
            pltpu.VMEM_SHARED((NPAD, CW), jnp.float32),
            pltpu.VMEM((2, AEB), jnp.int32),
            pltpu.VMEM((2, AEB), jnp.int32),
            pltpu.VMEM((2, AEB), jnp.int32),
            pltpu.VMEM((2, AEB, CW), jnp.float32),
            pltpu.SemaphoreType.DMA,
            pltpu.SemaphoreType.DMA,
            pltpu.SemaphoreType.DMA,
            pltpu.SemaphoreType.DMA,
            pltpu.SemaphoreType.DMA,
            pltpu.SemaphoreType.DMA,
        ],
    )(_agg_body)
    return k(hp, hp4, srcp, dstp)


BM = 1024


def _dis(deg_ref):
    return lax.rsqrt(deg_ref[0, :, 0:1] + deg_ref[1, :, 0:1] + 1.0)


def _tc1_body(x_ref, w_ref, deg_ref, o_ref):
    dis = _dis(deg_ref)
    h = jnp.dot(x_ref[...], w_ref[...], preferred_element_type=jnp.float32)
    o_ref[...] = h * dis


def _tc1(xpad, W1, deg):
    return pl.pallas_call(
        _tc1_body,
        grid=(NPAD // BM,),
        in_specs=[
            pl.BlockSpec((BM, D), lambda m: (m, 0)),
            pl.BlockSpec((D, D), lambda m: (0, 0)),
            pl.BlockSpec((2, BM, DEGW), lambda m: (0, m, 0)),
        ],
        out_specs=pl.BlockSpec((BM, D), lambda m: (m, 0)),
        out_shape=jax.ShapeDtypeStruct((NPAD, D), jnp.float32),
    )(xpad, W1, deg)


def _tc2_body(a_ref, w_ref, deg_ref, b_ref, o_ref):
    dis = _dis(deg_ref)
    h = jnp.maximum(a_ref[...] * dis + b_ref[...], 0.0)
    o_ref[...] = jnp.dot(h, w_ref[...], preferred_element_type=jnp.float32) * dis


def _tc2(acc1, W2, deg, b1):
    return pl.pallas_call(
        _tc2_body,
        grid=(NPAD // BM,),
        in_specs=[
            pl.BlockSpec((BM, D), lambda m: (m, 0)),
            pl.BlockSpec((D, D), lambda m: (0, 0)),
            pl.BlockSpec((2, BM, DEGW), lambda m: (0, m, 0)),
            pl.BlockSpec((1, D), lambda m: (0, 0)),
        ],
        out_specs=pl.BlockSpec((BM, D), lambda m: (m, 0)),
        out_shape=jax.ShapeDtypeStruct((NPAD, D), jnp.float32),
    )(acc1, W2, deg, b1)


def _tc3_body(a_ref, deg_ref, b_ref, o_ref):
    dis = _dis(deg_ref)
    o_ref[...] = a_ref[...] * dis + b_ref[...]


def _tc3(acc2, deg, b2):
    return pl.pallas_call(
        _tc3_body,
        grid=(NPAD // BM,),
        in_specs=[
            pl.BlockSpec((BM, D), lambda m: (m, 0)),
            pl.BlockSpec((2, BM, DEGW), lambda m: (0, m, 0)),
            pl.BlockSpec((1, D), lambda m: (0, 0)),
        ],
        out_specs=pl.BlockSpec((BM, D), lambda m: (m, 0)),
        out_shape=jax.ShapeDtypeStruct((NPAD, D), jnp.float32),
    )(acc2, deg, b2)


def kernel(x, edge_index, W1, b1, W2, b2):
    ei = edge_index.astype(jnp.int32)
    srcp = jnp.concatenate([ei[0], jnp.zeros((EPAD - E,), jnp.int32)])
    dstp = jnp.concatenate([ei[1], jnp.full((EPAD - E,), N, jnp.int32)])
    xpad = jnp.pad(x, ((0, NPAD - N), (0, 0)))
    ones_b = jnp.ones((EB, DEGW), jnp.float32)
    zeros_b = jnp.zeros((NPAD, DEGW), jnp.float32)

    deg = _sc_deg(dstp, ones_b, zeros_b)
    h1p = _tc1(xpad, W1, deg)
    acc1 = _sc_agg(h1p, srcp, dstp)
    h2p = _tc2(acc1, W2, deg, b1.reshape(1, D))
    acc2 = _sc_agg(h2p, srcp, dstp)
    out = _tc3(acc2, deg, b2.reshape(1, D))
    return out[:N]
